# baseline (speedup 1.0000x reference)
import functools

import jax
import jax.numpy as jnp
from jax import lax
from jax.experimental import pallas as pl
from jax.experimental.pallas import tpu as pltpu
from jax.experimental.pallas import tpu_sc as plsc

_NC = 2
_NS = 16
_NW = _NC * _NS
_L = 16
_C = 64


def _score_kernel(B, K):
    b_per_w = B // _NW
    n_chunks = b_per_w // _C
    mesh = plsc.VectorSubcoreMesh(core_axis_name="c", subcore_axis_name="s")

    @functools.partial(
        pl.kernel,
        out_type=jax.ShapeDtypeStruct((B,), jnp.float32),
        mesh=mesh,
        compiler_params=pltpu.CompilerParams(
            needs_layout_passes=False, use_tc_tiling_on_sc=False),
        scratch_types=[
            pltpu.VMEM((b_per_w,), jnp.int32),
            pltpu.VMEM((b_per_w,), jnp.int32),
            pltpu.VMEM((b_per_w,), jnp.int32),
            pltpu.VMEM((_C, K), jnp.float32),
            pltpu.VMEM((_C, K // 2), jnp.int32),
            pltpu.VMEM((_C, K), jnp.float32),
            pltpu.VMEM((_C, K), jnp.float32),
            pltpu.VMEM((_C, K // 2), jnp.int32),
            pltpu.VMEM((_C, K), jnp.float32),
            pltpu.VMEM((b_per_w,), jnp.float32),
            pltpu.VMEM((_L * _L,), jnp.float32),
            pltpu.SemaphoreType.DMA,
            pltpu.SemaphoreType.DMA,
        ],
    )
    def k(subj_hbm, rel_hbm, obj_hbm, ent_hbm, rel_emb_hbm, out_hbm,
          si, ri, oi, sbuf0, pbuf0, obuf0, sbuf1, pbuf1, obuf1,
          scores, tmp, sem0, sem1):
        wid = lax.axis_index("s") * _NC + lax.axis_index("c")
        base = wid * b_per_w
        pltpu.sync_copy(subj_hbm.at[pl.ds(base, b_per_w)], si)
        pltpu.sync_copy(rel_hbm.at[pl.ds(base, b_per_w)], ri)
        pltpu.sync_copy(obj_hbm.at[pl.ds(base, b_per_w)], oi)

        bufs = ((sbuf0, pbuf0, obuf0, sem0), (sbuf1, pbuf1, obuf1, sem1))

        def start(ci, bset):
            sb, pb, ob, sem = bset
            off = ci * _C
            return (
                pltpu.async_copy(ent_hbm.at[si.at[pl.ds(off, _C)]], sb, sem),
                pltpu.async_copy(rel_emb_hbm.at[ri.at[pl.ds(off, _C)]], pb, sem),
                pltpu.async_copy(ent_hbm.at[oi.at[pl.ds(off, _C)]], ob, sem),
            )

        def wait_set(bset):
            sb, pb, ob, sem = bset
            dummy = ent_hbm.at[pl.ds(0, _C)]
            dummy_p = rel_emb_hbm.at[pl.ds(0, _C)]
            pltpu.make_async_copy(dummy, sb, sem).wait()
            pltpu.make_async_copy(dummy_p, pb, sem).wait()
            pltpu.make_async_copy(dummy, ob, sem).wait()

        col_iota = lax.iota(jnp.int32, _L) * _L

        def compute(ci, bset):
            sb, pb, ob, _ = bset
            off = ci * _C

            def group_body(g, c2):
                for j in range(_L):
                    r = g * _L + j
                    acc = jnp.zeros((_L,), jnp.float32)
                    for m in range(K // (2 * _L)):
                        w = pb[r, pl.ds(m * _L, _L)]
                        plo = lax.bitcast_convert_type(
                            lax.shift_left(w, 16), jnp.float32)
                        phi = lax.bitcast_convert_type(
                            w & jnp.int32(-65536), jnp.float32)
                        s0 = sb[r, pl.ds((2 * m) * _L, _L)]
                        o0 = ob[r, pl.ds((2 * m) * _L, _L)]
                        s1 = sb[r, pl.ds((2 * m + 1) * _L, _L)]
                        o1 = ob[r, pl.ds((2 * m + 1) * _L, _L)]
                        acc = acc + s0 * plo * o0 + s1 * phi * o1
                    tmp[pl.ds(j * _L, _L)] = acc
                score_vec = jnp.zeros((_L,), jnp.float32)
                for l in range(_L):
                    score_vec = score_vec + plsc.load_gather(tmp, [col_iota + l])
                scores[pl.ds(off + g * _L, _L)] = score_vec
                return c2

            lax.fori_loop(0, _C // _L, group_body, 0)

        start(0, bufs[0])

        def pair_body(i, carry):
            ci0 = 2 * i
            h1 = start(ci0 + 1, bufs[1])
            wait_set(bufs[0])
            compute(ci0, bufs[0])
            start(ci0 + 2, bufs[0])
            for h in h1:
                h.wait()
            compute(ci0 + 1, bufs[1])
            return carry

        lax.fori_loop(0, n_chunks // 2 - 1, pair_body, 0)
        last = n_chunks - 2
        h1 = start(last + 1, bufs[1])
        wait_set(bufs[0])
        compute(last, bufs[0])
        for h in h1:
            h.wait()
        compute(last + 1, bufs[1])

        pltpu.sync_copy(scores, out_hbm.at[pl.ds(base, b_per_w)])

    return k


def kernel(inputs, ent_emb, rel_emb):
    B = inputs.shape[0]
    K = ent_emb.shape[1]
    subj = inputs[:, 0]
    rel = inputs[:, 1]
    obj = inputs[:, 2]
    rbits = jax.lax.bitcast_convert_type(
        rel_emb.astype(jnp.bfloat16), jnp.uint16).astype(jnp.uint32)
    r4 = rbits.reshape(rel_emb.shape[0], K // 32, 2, 16)
    packed = jax.lax.bitcast_convert_type(
        r4[:, :, 0, :] | (r4[:, :, 1, :] << 16), jnp.int32)
    packed = packed.reshape(rel_emb.shape[0], K // 2)
    return _score_kernel(B, K)(subj, rel, obj, ent_emb, packed)


# speedup vs baseline: 1.0618x; 1.0618x over previous
import functools

import jax
import jax.numpy as jnp
from jax import lax
from jax.experimental import pallas as pl
from jax.experimental.pallas import tpu as pltpu
from jax.experimental.pallas import tpu_sc as plsc

_NC = 2
_NS = 16
_NW = _NC * _NS
_L = 16
_C = 64


def _score_kernel(B, K):
    b_per_w = B // _NW
    n_chunks = b_per_w // _C
    mesh = plsc.VectorSubcoreMesh(core_axis_name="c", subcore_axis_name="s")

    @functools.partial(
        pl.kernel,
        out_type=jax.ShapeDtypeStruct((B,), jnp.float32),
        mesh=mesh,
        compiler_params=pltpu.CompilerParams(
            needs_layout_passes=False, use_tc_tiling_on_sc=False),
        scratch_types=[
            pltpu.VMEM((b_per_w,), jnp.int32),
            pltpu.VMEM((b_per_w,), jnp.int32),
            pltpu.VMEM((b_per_w,), jnp.int32),
            pltpu.VMEM((_C, K), jnp.float32),
            pltpu.VMEM((_C, K // 2), jnp.int32),
            pltpu.VMEM((_C, K), jnp.float32),
            pltpu.VMEM((_C, K), jnp.float32),
            pltpu.VMEM((_C, K // 2), jnp.int32),
            pltpu.VMEM((_C, K), jnp.float32),
            pltpu.VMEM((b_per_w,), jnp.float32),
            pltpu.VMEM((_L * _L,), jnp.float32),
            pltpu.SemaphoreType.DMA,
            pltpu.SemaphoreType.DMA,
        ],
    )
    def k(subj_hbm, rel_hbm, obj_hbm, ent_hbm, rel_emb_hbm, out_hbm,
          si, ri, oi, sbuf0, pbuf0, obuf0, sbuf1, pbuf1, obuf1,
          scores, tmp, sem0, sem1):
        wid = lax.axis_index("s") * _NC + lax.axis_index("c")
        base = wid * b_per_w
        hs = pltpu.async_copy(subj_hbm.at[pl.ds(base, b_per_w)], si, sem0)
        hr = pltpu.async_copy(rel_hbm.at[pl.ds(base, b_per_w)], ri, sem0)
        ho = pltpu.async_copy(obj_hbm.at[pl.ds(base, b_per_w)], oi, sem0)
        hs.wait()
        hr.wait()
        ho.wait()

        bufs = ((sbuf0, pbuf0, obuf0, sem0), (sbuf1, pbuf1, obuf1, sem1))

        def start(ci, bset):
            sb, pb, ob, sem = bset
            off = ci * _C
            return (
                pltpu.async_copy(ent_hbm.at[si.at[pl.ds(off, _C)]], sb, sem),
                pltpu.async_copy(rel_emb_hbm.at[ri.at[pl.ds(off, _C)]], pb, sem),
                pltpu.async_copy(ent_hbm.at[oi.at[pl.ds(off, _C)]], ob, sem),
            )

        def wait_set(bset):
            sb, pb, ob, sem = bset
            dummy = ent_hbm.at[pl.ds(0, _C)]
            dummy_p = rel_emb_hbm.at[pl.ds(0, _C)]
            pltpu.make_async_copy(dummy, sb, sem).wait()
            pltpu.make_async_copy(dummy_p, pb, sem).wait()
            pltpu.make_async_copy(dummy, ob, sem).wait()

        col_iota = lax.iota(jnp.int32, _L) * _L

        def compute(ci, bset):
            sb, pb, ob, _ = bset
            off = ci * _C

            def group_body(g, c2):
                for j in range(_L):
                    r = g * _L + j
                    acc = jnp.zeros((_L,), jnp.float32)
                    for m in range(K // (2 * _L)):
                        w = pb[r, pl.ds(m * _L, _L)]
                        plo = lax.bitcast_convert_type(
                            lax.shift_left(w, 16), jnp.float32)
                        phi = lax.bitcast_convert_type(
                            w & jnp.int32(-65536), jnp.float32)
                        s0 = sb[r, pl.ds((2 * m) * _L, _L)]
                        o0 = ob[r, pl.ds((2 * m) * _L, _L)]
                        s1 = sb[r, pl.ds((2 * m + 1) * _L, _L)]
                        o1 = ob[r, pl.ds((2 * m + 1) * _L, _L)]
                        acc = acc + s0 * plo * o0 + s1 * phi * o1
                    tmp[pl.ds(j * _L, _L)] = acc
                score_vec = jnp.zeros((_L,), jnp.float32)
                for l in range(_L):
                    score_vec = score_vec + plsc.load_gather(tmp, [col_iota + l])
                scores[pl.ds(off + g * _L, _L)] = score_vec
                return c2

            lax.fori_loop(0, _C // _L, group_body, 0)

        start(0, bufs[0])

        def pair_body(i, carry):
            ci0 = 2 * i
            h1 = start(ci0 + 1, bufs[1])
            wait_set(bufs[0])
            compute(ci0, bufs[0])
            nxt = lax.min(ci0 + 2, n_chunks - 2)
            start(nxt, bufs[0])
            for h in h1:
                h.wait()
            compute(ci0 + 1, bufs[1])
            return carry

        lax.fori_loop(0, n_chunks // 2, pair_body, 0)
        wait_set(bufs[0])

        pltpu.sync_copy(scores, out_hbm.at[pl.ds(base, b_per_w)])

    return k


def kernel(inputs, ent_emb, rel_emb):
    B = inputs.shape[0]
    K = ent_emb.shape[1]
    subj = inputs[:, 0]
    rel = inputs[:, 1]
    obj = inputs[:, 2]
    rbits = jax.lax.bitcast_convert_type(
        rel_emb.astype(jnp.bfloat16), jnp.uint16).astype(jnp.uint32)
    r4 = rbits.reshape(rel_emb.shape[0], K // 32, 2, 16)
    packed = jax.lax.bitcast_convert_type(
        r4[:, :, 0, :] | (r4[:, :, 1, :] << 16), jnp.int32)
    packed = packed.reshape(rel_emb.shape[0], K // 2)
    return _score_kernel(B, K)(subj, rel, obj, ent_emb, packed)
